# baseline (device time: 31281 ns/iter reference)
import jax
import jax.numpy as jnp
from jax import lax
from jax.experimental import pallas as pl
from jax.experimental.pallas import tpu as pltpu

N_DEV = 4


def kernel(x, router_W, route_idx, expert_W, shared_W):
    n, d = x.shape
    e_loc, _, h_dim = expert_W.shape
    n_exp = e_loc * N_DEV

    def body(x_ref, rw_ref, idx_ref, ew_ref, sw_ref, out_ref,
             comm_ref, send_sems, recv_sems):
        my = lax.axis_index("i")
        left = lax.rem(my + N_DEV - 1, N_DEV)
        right = lax.rem(my + 1, N_DEV)

        barrier_sem = pltpu.get_barrier_semaphore()
        for nbr in (left, right):
            pl.semaphore_signal(
                barrier_sem, inc=1,
                device_id=(nbr,), device_id_type=pl.DeviceIdType.MESH,
            )
        pl.semaphore_wait(barrier_sem, 2)

        x_f32 = x_ref[...]
        x_bf = x_f32.astype(jnp.bfloat16)

        scores = jnp.dot(x_f32, rw_ref[...], preferred_element_type=jnp.float32)
        m = jnp.max(scores, axis=-1, keepdims=True)
        e = jnp.exp(scores - m)
        probs = e / jnp.sum(e, axis=-1, keepdims=True)
        route = idx_ref[...]

        acc = jnp.dot(x_bf, sw_ref[...].astype(jnp.bfloat16),
                      preferred_element_type=jnp.float32)

        comm_ref[0] = ew_ref[...].astype(jnp.bfloat16)

        iota_e = lax.broadcasted_iota(jnp.int32, (n, n_exp), 1)

        def contrib(slot, origin, acc):
            for j in range(e_loc):
                eid = origin * e_loc + j
                p_e = jnp.sum(jnp.where(iota_e == eid, probs, 0.0),
                              axis=1, keepdims=True)
                coeff = jnp.where(route == eid, p_e, 0.0)
                y = jnp.dot(x_bf, comm_ref[slot, j],
                            preferred_element_type=jnp.float32)
                acc = acc + coeff * y
            return acc

        for h in range(N_DEV - 1):
            s, r = h % 2, (h + 1) % 2
            rdma = pltpu.make_async_remote_copy(
                src_ref=comm_ref.at[s],
                dst_ref=comm_ref.at[r],
                send_sem=send_sems.at[s],
                recv_sem=recv_sems.at[r],
                device_id=(right,),
                device_id_type=pl.DeviceIdType.MESH,
            )
            rdma.start()
            origin = lax.rem(my + N_DEV - h, N_DEV) if h else my
            acc = contrib(s, origin, acc)
            rdma.wait()

        acc = contrib((N_DEV - 1) % 2, right, acc)
        out_ref[...] = acc

    return pl.pallas_call(
        body,
        out_shape=jax.ShapeDtypeStruct((n, h_dim), jnp.float32),
        in_specs=[pl.BlockSpec(memory_space=pltpu.VMEM)] * 5,
        out_specs=pl.BlockSpec(memory_space=pltpu.VMEM),
        scratch_shapes=[
            pltpu.VMEM((2, e_loc, d, h_dim), jnp.bfloat16),
            pltpu.SemaphoreType.DMA((2,)),
            pltpu.SemaphoreType.DMA((2,)),
        ],
        compiler_params=pltpu.CompilerParams(collective_id=0),
    )(x, router_W, route_idx, expert_W, shared_W)


# device time: 20680 ns/iter; 1.5126x vs baseline; 1.5126x over previous
import jax
import jax.numpy as jnp
from jax import lax
from jax.experimental import pallas as pl
from jax.experimental.pallas import tpu as pltpu

N_DEV = 4

_A = 0
_B = 1
_C = 2
_D = 3


def kernel(x, router_W, route_idx, expert_W, shared_W):
    n, d = x.shape
    e_loc, _, h_dim = expert_W.shape
    n_exp = e_loc * N_DEV

    expert_bf = expert_W.astype(jnp.bfloat16)
    shared_bf = shared_W.astype(jnp.bfloat16)

    def body(x_ref, rw_ref, idx_ref, ew_ref, sw_ref, out_ref,
             comm_ref, send_sems, recv_sems):
        my = lax.axis_index("i")
        left = lax.rem(my + N_DEV - 1, N_DEV)
        right = lax.rem(my + 1, N_DEV)
        diag = lax.rem(my + 2, N_DEV)

        barrier_sem = pltpu.get_barrier_semaphore()
        for nbr in (left, right):
            pl.semaphore_signal(
                barrier_sem, inc=1,
                device_id=(nbr,), device_id_type=pl.DeviceIdType.MESH,
            )
        pl.semaphore_wait(barrier_sem, 2)

        def copy(idx, src, dst, dev):
            return pltpu.make_async_remote_copy(
                src_ref=src, dst_ref=dst,
                send_sem=send_sems.at[idx], recv_sem=recv_sems.at[idx],
                device_id=(dev,), device_id_type=pl.DeviceIdType.MESH,
            )

        rdma_a = copy(_A, ew_ref, comm_ref.at[0], right)
        rdma_b = copy(_B, ew_ref, comm_ref.at[1], left)
        rdma_a.start()
        rdma_b.start()

        x_f32 = x_ref[...]
        x_bf = x_f32.astype(jnp.bfloat16)

        scores = jnp.dot(x_f32, rw_ref[...], preferred_element_type=jnp.float32)
        m = jnp.max(scores, axis=-1, keepdims=True)
        e = jnp.exp(scores - m)
        probs = e / jnp.sum(e, axis=-1, keepdims=True)
        route = idx_ref[...]

        acc = jnp.dot(x_bf, sw_ref[...], preferred_element_type=jnp.float32)

        iota_e = lax.broadcasted_iota(jnp.int32, (n, n_exp), 1)

        def contrib(w_ref, origin, acc):
            for j in range(e_loc):
                eid = origin * e_loc + j
                p_e = jnp.sum(jnp.where(iota_e == eid, probs, 0.0),
                              axis=1, keepdims=True)
                coeff = jnp.where(route == eid, p_e, 0.0)
                y = jnp.dot(x_bf, w_ref[j],
                            preferred_element_type=jnp.float32)
                acc = acc + coeff * y
            return acc

        acc = contrib(ew_ref, my, acc)

        rdma_a.wait_recv()
        rdma_c = copy(_C, comm_ref.at[0, 0], comm_ref.at[2, 0], right)
        rdma_c.start()
        acc = contrib(comm_ref.at[0], left, acc)

        rdma_b.wait_recv()
        rdma_d = copy(_D, comm_ref.at[1, 1], comm_ref.at[2, 1], left)
        rdma_d.start()
        acc = contrib(comm_ref.at[1], right, acc)

        rdma_c.wait_recv()
        rdma_d.wait_recv()
        acc = contrib(comm_ref.at[2], diag, acc)

        out_ref[...] = acc

        rdma_a.wait_send()
        rdma_b.wait_send()
        rdma_c.wait_send()
        rdma_d.wait_send()

    return pl.pallas_call(
        body,
        out_shape=jax.ShapeDtypeStruct((n, h_dim), jnp.float32),
        in_specs=[pl.BlockSpec(memory_space=pltpu.VMEM)] * 5,
        out_specs=pl.BlockSpec(memory_space=pltpu.VMEM),
        scratch_shapes=[
            pltpu.VMEM((3, e_loc, d, h_dim), jnp.bfloat16),
            pltpu.SemaphoreType.DMA((4,)),
            pltpu.SemaphoreType.DMA((4,)),
        ],
        compiler_params=pltpu.CompilerParams(collective_id=0),
    )(x, router_W, route_idx, expert_bf, shared_bf)


# device time: 19634 ns/iter; 1.5932x vs baseline; 1.0533x over previous
import jax
import jax.numpy as jnp
from jax import lax
from jax.experimental import pallas as pl
from jax.experimental.pallas import tpu as pltpu

N_DEV = 4

_A0, _A1, _B0, _B1, _C, _D = range(6)


def kernel(x, router_W, route_idx, expert_W, shared_W):
    n, d = x.shape
    e_loc, _, h_dim = expert_W.shape
    n_exp = e_loc * N_DEV

    def body(x_ref, rw_ref, idx_ref, ew_ref, sw_ref, out_ref,
             own_ref, comm_ref, send_sems, recv_sems):
        my = lax.axis_index("i")
        left = lax.rem(my + N_DEV - 1, N_DEV)
        right = lax.rem(my + 1, N_DEV)
        diag = lax.rem(my + 2, N_DEV)

        own_ref[0] = ew_ref[0].astype(jnp.bfloat16)
        own_ref[1] = ew_ref[1].astype(jnp.bfloat16)

        barrier_sem = pltpu.get_barrier_semaphore()
        for nbr in (left, right):
            pl.semaphore_signal(
                barrier_sem, inc=1,
                device_id=(nbr,), device_id_type=pl.DeviceIdType.MESH,
            )
        pl.semaphore_wait(barrier_sem, 2)

        def copy(idx, src, dst, dev):
            return pltpu.make_async_remote_copy(
                src_ref=src, dst_ref=dst,
                send_sem=send_sems.at[idx], recv_sem=recv_sems.at[idx],
                device_id=(dev,), device_id_type=pl.DeviceIdType.MESH,
            )

        rdma_a0 = copy(_A0, own_ref.at[0], comm_ref.at[0, 0], right)
        rdma_b1 = copy(_B1, own_ref.at[1], comm_ref.at[1, 1], left)
        rdma_a1 = copy(_A1, own_ref.at[1], comm_ref.at[0, 1], right)
        rdma_b0 = copy(_B0, own_ref.at[0], comm_ref.at[1, 0], left)
        rdma_a0.start()
        rdma_b1.start()
        rdma_a1.start()
        rdma_b0.start()

        x_f32 = x_ref[...]
        scores = jnp.dot(x_f32, rw_ref[...], preferred_element_type=jnp.float32)
        x_bf = x_f32.astype(jnp.bfloat16)

        m = jnp.max(scores, axis=-1, keepdims=True)
        e = jnp.exp(scores - m)
        probs = e / jnp.sum(e, axis=-1, keepdims=True)
        route = idx_ref[...]

        acc = jnp.dot(x_bf, sw_ref[...].astype(jnp.bfloat16),
                      preferred_element_type=jnp.float32)

        iota_e = lax.broadcasted_iota(jnp.int32, (n, n_exp), 1)

        def contrib(w_half_ref, eid, acc):
            p_e = jnp.sum(jnp.where(iota_e == eid, probs, 0.0),
                          axis=1, keepdims=True)
            coeff = jnp.where(route == eid, p_e, 0.0)
            y = jnp.dot(x_bf, w_half_ref[...],
                        preferred_element_type=jnp.float32)
            return acc + coeff * y

        for j in range(e_loc):
            acc = contrib(own_ref.at[j], my * e_loc + j, acc)

        rdma_a0.wait_recv()
        rdma_c = copy(_C, comm_ref.at[0, 0], comm_ref.at[2, 0], right)
        rdma_c.start()
        rdma_b1.wait_recv()
        rdma_d = copy(_D, comm_ref.at[1, 1], comm_ref.at[2, 1], left)
        rdma_d.start()

        acc = contrib(comm_ref.at[0, 0], left * e_loc, acc)
        acc = contrib(comm_ref.at[1, 1], right * e_loc + 1, acc)

        rdma_a1.wait_recv()
        acc = contrib(comm_ref.at[0, 1], left * e_loc + 1, acc)
        rdma_b0.wait_recv()
        acc = contrib(comm_ref.at[1, 0], right * e_loc, acc)

        rdma_c.wait_recv()
        acc = contrib(comm_ref.at[2, 0], diag * e_loc, acc)
        rdma_d.wait_recv()
        acc = contrib(comm_ref.at[2, 1], diag * e_loc + 1, acc)

        out_ref[...] = acc

        for r in (rdma_a0, rdma_b1, rdma_a1, rdma_b0, rdma_c, rdma_d):
            r.wait_send()

    return pl.pallas_call(
        body,
        out_shape=jax.ShapeDtypeStruct((n, h_dim), jnp.float32),
        in_specs=[pl.BlockSpec(memory_space=pltpu.VMEM)] * 5,
        out_specs=pl.BlockSpec(memory_space=pltpu.VMEM),
        scratch_shapes=[
            pltpu.VMEM((e_loc, d, h_dim), jnp.bfloat16),
            pltpu.VMEM((3, e_loc, d, h_dim), jnp.bfloat16),
            pltpu.SemaphoreType.DMA((6,)),
            pltpu.SemaphoreType.DMA((6,)),
        ],
        compiler_params=pltpu.CompilerParams(collective_id=0),
    )(x, router_W, route_idx, expert_W, shared_W)
